# baseline (speedup 1.0000x reference)
import functools

import jax
import jax.numpy as jnp
from jax import lax
from jax.experimental import pallas as pl
from jax.experimental.pallas import tpu as pltpu
from jax.experimental.pallas import tpu_sc as plsc

N_TOK = 9216
K = 1024
D = 64
BLK = 1152
GRID = N_TOK // BLK
LOSS_SCALE = 1.25 / (N_TOK * D)


def _tc_body(x_ref, w_ref, idx_ref, loss_ref, wpad_ref):
    i = pl.program_id(0)
    x = x_ref[...]
    w = w_ref[...]
    xw = lax.dot_general(x, w, (((1,), (1,)), ((), ())),
                         preferred_element_type=jnp.float32)
    x2 = jnp.sum(x * x, axis=1, keepdims=True)
    w2 = jnp.sum(w * w, axis=1, keepdims=True).T
    d2 = x2 + w2 - 2.0 * xw
    dist = jnp.sqrt(jnp.maximum(d2, 0.0))
    m = jnp.min(dist, axis=1, keepdims=True)
    lanes = lax.broadcasted_iota(jnp.int32, (BLK, K), 1)
    idx = jnp.min(jnp.where(dist == m, lanes, K), axis=1, keepdims=True)
    idx_ref[...] = idx
    part = jnp.sum(m * m).reshape(1, 1)
    acc = jnp.where(i == 0, part, loss_ref[...] + part)
    loss_ref[...] = jnp.where(i == GRID - 1, acc * LOSS_SCALE, acc)

    @pl.when(i == 0)
    def _():
        wpad_ref[...] = jnp.concatenate([w, w], axis=1)


_tc_call = pl.pallas_call(
    _tc_body,
    grid=(GRID,),
    in_specs=[
        pl.BlockSpec((BLK, D), lambda i: (i, 0)),
        pl.BlockSpec((K, D), lambda i: (0, 0)),
    ],
    out_specs=[
        pl.BlockSpec((BLK, 1), lambda i: (i, 0)),
        pl.BlockSpec((1, 1), lambda i: (0, 0)),
        pl.BlockSpec((K, 2 * D), lambda i: (0, 0)),
    ],
    out_shape=[
        jax.ShapeDtypeStruct((N_TOK, 1), jnp.int32),
        jax.ShapeDtypeStruct((1, 1), jnp.float32),
        jax.ShapeDtypeStruct((K, 2 * D), jnp.float32),
    ],
)


_NC, _NS = 2, 16
_NW = _NC * _NS
_B_PER_W = N_TOK // _NW


DPAD = 128
_CHUNKS = 3
_CHUNK = _B_PER_W // _CHUNKS


@functools.lru_cache(maxsize=1)
def _make_sc_gather():
    mesh = plsc.VectorSubcoreMesh(
        core_axis_name="c", subcore_axis_name="s",
        num_cores=_NC, num_subcores=_NS,
    )

    @functools.partial(
        pl.kernel,
        mesh=mesh,
        out_type=jax.ShapeDtypeStruct((N_TOK, DPAD), jnp.float32),
        scratch_types=[
            pltpu.VMEM((_CHUNKS, _CHUNK), jnp.int32),
            pltpu.VMEM((_B_PER_W, DPAD), jnp.float32),
            pltpu.SemaphoreType.DMA,
        ],
    )
    def _sc_gather(table_hbm, idx_hbm, out_hbm, idx_v, rows_v, sem):
        wid = lax.axis_index("s") * _NC + lax.axis_index("c")
        base = wid * _B_PER_W
        pltpu.sync_copy(idx_hbm.at[wid], idx_v)
        copies = [
            pltpu.async_copy(
                table_hbm.at[idx_v.at[j]],
                rows_v.at[pl.ds(j * _CHUNK, _CHUNK)],
                sem,
            )
            for j in range(_CHUNKS)
        ]
        for cp in copies:
            cp.wait()
        pltpu.sync_copy(rows_v, out_hbm.at[pl.ds(base, _B_PER_W)])

    return _sc_gather


def kernel(x, W):
    idx2d, loss, W_pad = _tc_call(x, W)
    idx = idx2d.reshape(N_TOK)
    idx_rows = idx.reshape(_NW, _CHUNKS, _CHUNK)
    quantised_pad = _make_sc_gather()(W_pad, idx_rows)
    return quantised_pad[:, :D], loss[0, 0], idx


# speedup vs baseline: 1.0469x; 1.0469x over previous
import functools

import jax
import jax.numpy as jnp
from jax import lax
from jax.experimental import pallas as pl
from jax.experimental.pallas import tpu as pltpu
from jax.experimental.pallas import tpu_sc as plsc

N_TOK = 9216
K = 1024
D = 64
BLK = 1152
GRID = N_TOK // BLK
LOSS_SCALE = 1.25 / (N_TOK * D)


def _tc_body(x_ref, w_ref, idx_ref, loss_ref, wt_ref):
    i = pl.program_id(0)

    @pl.when(i == 0)
    def _():
        wt_ref[...] = w_ref[...].T

    x = x_ref[...]
    wt = wt_ref[...]
    xw = lax.dot_general(x, wt, (((1,), (0,)), ((), ())),
                         preferred_element_type=jnp.float32)
    x2 = jnp.sum(x * x, axis=1, keepdims=True)
    w2 = jnp.sum(wt * wt, axis=0, keepdims=True)
    d2 = x2 + w2 - 2.0 * xw
    dist = jnp.sqrt(jnp.maximum(d2, 0.0))
    m = jnp.min(dist, axis=1, keepdims=True)
    lanes = lax.broadcasted_iota(jnp.int32, (BLK, K), 1)
    idx = jnp.min(jnp.where(dist == m, lanes, K), axis=1, keepdims=True)
    idx_ref[...] = idx
    part = jnp.sum(m * m).reshape(1, 1)
    acc = jnp.where(i == 0, part, loss_ref[...] + part)
    loss_ref[...] = jnp.where(i == GRID - 1, acc * LOSS_SCALE, acc)


_tc_call = pl.pallas_call(
    _tc_body,
    grid=(GRID,),
    in_specs=[
        pl.BlockSpec((BLK, D), lambda i: (i, 0)),
        pl.BlockSpec((K, D), lambda i: (0, 0)),
    ],
    out_specs=[
        pl.BlockSpec((BLK, 1), lambda i: (i, 0)),
        pl.BlockSpec((1, 1), lambda i: (0, 0)),
    ],
    out_shape=[
        jax.ShapeDtypeStruct((N_TOK, 1), jnp.int32),
        jax.ShapeDtypeStruct((1, 1), jnp.float32),
    ],
    scratch_shapes=[pltpu.VMEM((D, K), jnp.float32)],
)


_NC, _NS = 2, 16
_NW = _NC * _NS
_B_PER_W = N_TOK // _NW
DPAD = 128
_CHUNKS = 3
_CHUNK = _B_PER_W // _CHUNKS


@functools.lru_cache(maxsize=1)
def _make_sc_gather():
    mesh = plsc.VectorSubcoreMesh(
        core_axis_name="c", subcore_axis_name="s",
        num_cores=_NC, num_subcores=_NS,
    )

    @functools.partial(
        pl.kernel,
        mesh=mesh,
        out_type=[
            jax.ShapeDtypeStruct((N_TOK, DPAD), jnp.float32),
            jax.ShapeDtypeStruct((N_TOK,), jnp.int32),
        ],
        scratch_types=[
            pltpu.VMEM((_CHUNKS, _CHUNK), jnp.int32),
            pltpu.VMEM((_B_PER_W, DPAD), jnp.float32),
            pltpu.SemaphoreType.DMA,
        ],
    )
    def _sc_gather(table_hbm, idx_hbm, out_hbm, idxout_hbm, idx_v, rows_v, sem):
        wid = lax.axis_index("s") * _NC + lax.axis_index("c")
        base = wid * _B_PER_W
        pltpu.sync_copy(idx_hbm.at[wid], idx_v)
        copies = [
            pltpu.async_copy(
                table_hbm.at[idx_v.at[j]],
                rows_v.at[pl.ds(j * _CHUNK, _CHUNK)],
                sem,
            )
            for j in range(_CHUNKS)
        ]
        for j in range(_CHUNKS):
            pltpu.sync_copy(idx_v.at[j],
                            idxout_hbm.at[pl.ds(base + j * _CHUNK, _CHUNK)])
        for cp in copies:
            cp.wait()
        pltpu.sync_copy(rows_v, out_hbm.at[pl.ds(base, _B_PER_W)])

    return _sc_gather


def kernel(x, W):
    idx2d, loss = _tc_call(x, W)
    W_pad = jnp.pad(W, ((0, 0), (0, DPAD - D)))
    idx_rows = idx2d.reshape(_NW, _CHUNKS, _CHUNK)
    quantised_pad, idx = _make_sc_gather()(W_pad, idx_rows)
    return quantised_pad[:, :D], loss[0, 0], idx


# speedup vs baseline: 1.0712x; 1.0232x over previous
import functools

import jax
import jax.numpy as jnp
from jax import lax
from jax.experimental import pallas as pl
from jax.experimental.pallas import tpu as pltpu
from jax.experimental.pallas import tpu_sc as plsc

N_TOK = 9216
K = 1024
D = 64
BLK = 1152
GRID = N_TOK // BLK
LOSS_SCALE = 1.25 / (N_TOK * D)


def _tc_body(x_ref, w_ref, idx_ref, loss_ref):
    i = pl.program_id(0)
    x = x_ref[...]
    w = w_ref[...]
    wx = lax.dot_general(w, x, (((1,), (1,)), ((), ())),
                         preferred_element_type=jnp.float32)
    x2 = jnp.sum(x * x, axis=1, keepdims=True).T
    w2 = jnp.sum(w * w, axis=1, keepdims=True)
    d2 = x2 + w2 - 2.0 * wx
    dist = jnp.sqrt(jnp.maximum(d2, 0.0))
    m = jnp.min(dist, axis=0, keepdims=True)
    rows = lax.broadcasted_iota(jnp.int32, (K, BLK), 0)
    idx = jnp.min(jnp.where(dist == m, rows, K), axis=0, keepdims=True)
    idx_ref[...] = idx.reshape(1, 1, BLK)
    part = jnp.sum(m * m).reshape(1, 1)
    acc = jnp.where(i == 0, part, loss_ref[...] + part)
    loss_ref[...] = jnp.where(i == GRID - 1, acc * LOSS_SCALE, acc)


_tc_call = pl.pallas_call(
    _tc_body,
    grid=(GRID,),
    in_specs=[
        pl.BlockSpec((BLK, D), lambda i: (i, 0)),
        pl.BlockSpec((K, D), lambda i: (0, 0)),
    ],
    out_specs=[
        pl.BlockSpec((1, 1, BLK), lambda i: (i, 0, 0)),
        pl.BlockSpec((1, 1), lambda i: (0, 0)),
    ],
    out_shape=[
        jax.ShapeDtypeStruct((GRID, 1, BLK), jnp.int32),
        jax.ShapeDtypeStruct((1, 1), jnp.float32),
    ],
)


_NC, _NS = 2, 16
_NW = _NC * _NS
_B_PER_W = N_TOK // _NW
DPAD = 128
_CHUNKS = 3
_CHUNK = _B_PER_W // _CHUNKS


@functools.lru_cache(maxsize=1)
def _make_sc_gather():
    mesh = plsc.VectorSubcoreMesh(
        core_axis_name="c", subcore_axis_name="s",
        num_cores=_NC, num_subcores=_NS,
    )

    @functools.partial(
        pl.kernel,
        mesh=mesh,
        out_type=[
            jax.ShapeDtypeStruct((N_TOK, DPAD), jnp.float32),
            jax.ShapeDtypeStruct((N_TOK,), jnp.int32),
        ],
        scratch_types=[
            pltpu.VMEM((_CHUNKS, _CHUNK), jnp.int32),
            pltpu.VMEM((_B_PER_W, DPAD), jnp.float32),
            pltpu.SemaphoreType.DMA,
        ],
    )
    def _sc_gather(table_hbm, idx_hbm, out_hbm, idxout_hbm, idx_v, rows_v, sem):
        wid = lax.axis_index("s") * _NC + lax.axis_index("c")
        base = wid * _B_PER_W
        pltpu.sync_copy(idx_hbm.at[wid], idx_v)
        copies = [
            pltpu.async_copy(
                table_hbm.at[idx_v.at[j]],
                rows_v.at[pl.ds(j * _CHUNK, _CHUNK)],
                sem,
            )
            for j in range(_CHUNKS)
        ]
        for j in range(_CHUNKS):
            pltpu.sync_copy(idx_v.at[j],
                            idxout_hbm.at[pl.ds(base + j * _CHUNK, _CHUNK)])
        for cp in copies:
            cp.wait()
        pltpu.sync_copy(rows_v, out_hbm.at[pl.ds(base, _B_PER_W)])

    return _sc_gather


def kernel(x, W):
    idx3d, loss = _tc_call(x, W)
    W_pad = jnp.pad(W, ((0, 0), (0, DPAD - D)))
    idx_rows = idx3d.reshape(_NW, _CHUNKS, _CHUNK)
    quantised_pad, idx = _make_sc_gather()(W_pad, idx_rows)
    return quantised_pad[:, :D], loss[0, 0], idx


# speedup vs baseline: 1.1091x; 1.0354x over previous
import functools

import jax
import jax.numpy as jnp
from jax import lax
from jax.experimental import pallas as pl
from jax.experimental.pallas import tpu as pltpu
from jax.experimental.pallas import tpu_sc as plsc

N_TOK = 9216
K = 1024
D = 64
BLK = 1152
GRID = N_TOK // BLK
LOSS_SCALE = 1.25 / (N_TOK * D)


def _tc_body(x_ref, w_ref, idx_ref, loss_ref):
    w = w_ref[...]
    w2 = jnp.sum(w * w, axis=1, keepdims=True)
    rows = lax.broadcasted_iota(jnp.int32, (K, BLK), 0)

    def step(i, acc):
        x = x_ref[pl.ds(i * BLK, BLK), :]
        wx = lax.dot_general(w, x, (((1,), (1,)), ((), ())),
                             preferred_element_type=jnp.float32)
        x2 = jnp.sum(x * x, axis=1, keepdims=True).T
        d2 = x2 + w2 - 2.0 * wx
        d2c = jnp.maximum(d2, 0.0)
        m2 = jnp.min(d2c, axis=0, keepdims=True)
        eqm = d2c == m2
        idx_fast = jnp.min(jnp.where(eqm, rows, K), axis=0, keepdims=True)
        idx_ref[pl.ds(i, 1), :, :] = idx_fast.reshape(1, 1, BLK)
        m2b = jnp.min(jnp.where(eqm, jnp.inf, d2c), axis=0, keepdims=True)
        tie_risk = jnp.any(m2b <= m2 * (1.0 + 1e-6))

        @pl.when(tie_risk)
        def _():
            dist = jnp.sqrt(d2c)
            m = jnp.min(dist, axis=0, keepdims=True)
            idx = jnp.min(jnp.where(dist == m, rows, K), axis=0, keepdims=True)
            idx_ref[pl.ds(i, 1), :, :] = idx.reshape(1, 1, BLK)

        return acc + jnp.sum(m2)

    total = lax.fori_loop(0, GRID, step, jnp.float32(0.0))
    loss_ref[...] = (total * LOSS_SCALE).reshape(1, 1)


_tc_call = pl.pallas_call(
    _tc_body,
    grid=(1,),
    in_specs=[
        pl.BlockSpec((N_TOK, D), lambda i: (0, 0)),
        pl.BlockSpec((K, D), lambda i: (0, 0)),
    ],
    out_specs=[
        pl.BlockSpec((GRID, 1, BLK), lambda i: (0, 0, 0)),
        pl.BlockSpec((1, 1), lambda i: (0, 0)),
    ],
    out_shape=[
        jax.ShapeDtypeStruct((GRID, 1, BLK), jnp.int32),
        jax.ShapeDtypeStruct((1, 1), jnp.float32),
    ],
)


_NC, _NS = 2, 16
_NW = _NC * _NS
_B_PER_W = N_TOK // _NW
DPAD = 128
_CHUNKS = 3
_CHUNK = _B_PER_W // _CHUNKS


@functools.lru_cache(maxsize=1)
def _make_sc_gather():
    mesh = plsc.VectorSubcoreMesh(
        core_axis_name="c", subcore_axis_name="s",
        num_cores=_NC, num_subcores=_NS,
    )

    @functools.partial(
        pl.kernel,
        mesh=mesh,
        out_type=[
            jax.ShapeDtypeStruct((N_TOK, DPAD), jnp.float32),
            jax.ShapeDtypeStruct((N_TOK,), jnp.int32),
        ],
        scratch_types=[
            pltpu.VMEM((_CHUNKS, _CHUNK), jnp.int32),
            pltpu.VMEM((_B_PER_W, DPAD), jnp.float32),
            pltpu.SemaphoreType.DMA,
            pltpu.SemaphoreType.DMA,
            pltpu.SemaphoreType.DMA,
        ],
    )
    def _sc_gather(table_hbm, idx_hbm, out_hbm, idxout_hbm, idx_v, rows_v,
                   sem, sem2, sem3):
        wid = lax.axis_index("s") * _NC + lax.axis_index("c")
        base = wid * _B_PER_W
        pltpu.sync_copy(idx_hbm.at[wid], idx_v)
        copies = [
            pltpu.async_copy(
                table_hbm.at[idx_v.at[j]],
                rows_v.at[pl.ds(j * _CHUNK, _CHUNK)],
                sem,
            )
            for j in range(_CHUNKS)
        ]
        iocopies = [
            pltpu.async_copy(
                idx_v.at[j],
                idxout_hbm.at[pl.ds(base + j * _CHUNK, _CHUNK)],
                sem2,
            )
            for j in range(_CHUNKS)
        ]
        outcopies = []
        for j in range(_CHUNKS):
            copies[j].wait()
            outcopies.append(pltpu.async_copy(
                rows_v.at[pl.ds(j * _CHUNK, _CHUNK)],
                out_hbm.at[pl.ds(base + j * _CHUNK, _CHUNK)],
                sem3,
            ))
        for cp in iocopies:
            cp.wait()
        for cp in outcopies:
            cp.wait()

    return _sc_gather


def kernel(x, W):
    idx3d, loss = _tc_call(x, W)
    W_pad = jnp.pad(W, ((0, 0), (0, DPAD - D)))
    idx_rows = idx3d.reshape(_NW, _CHUNKS, _CHUNK)
    quantised_pad, idx = _make_sc_gather()(W_pad, idx_rows)
    return quantised_pad[:, :D], loss[0, 0], idx
